# baseline (speedup 1.0000x reference)
import functools

import jax
import jax.numpy as jnp
from jax import lax
from jax.experimental import pallas as pl
from jax.experimental.pallas import tpu as pltpu
from jax.experimental.pallas import tpu_sc as plsc

_N = 10000
_E = 320000
_D = 128
_NG = 128
_NC = 10
_BN_EPS = 1e-5

_NCORE = 2
_NSUB = 16
_NW = _NCORE * _NSUB
_CHUNK = 80
_CPT = _E // _CHUNK // _NW
_RPT = 624
_ZROWS = 48


def _sc_agg(h, src3d, dst3d):
    mesh = plsc.VectorSubcoreMesh(core_axis_name="c", subcore_axis_name="s")

    @functools.partial(
        pl.kernel,
        out_type=jax.ShapeDtypeStruct((2 * _N, _D), jnp.float32),
        mesh=mesh,
        compiler_params=pltpu.CompilerParams(use_tc_tiling_on_sc=False),
        scratch_types=[
            pltpu.VMEM_SHARED((_N, _D), jnp.float32),
            pltpu.VMEM((1, _CPT * _CHUNK), jnp.int32),
            pltpu.VMEM((_CPT, _CHUNK), jnp.int32),
            pltpu.VMEM((_CHUNK, _D), jnp.float32),
            pltpu.VMEM((_CHUNK, _D), jnp.float32),
            pltpu.VMEM((_CHUNK, _D), jnp.float32),
            pltpu.SemaphoreType.DMA,
            pltpu.SemaphoreType.DMA,
            pltpu.SemaphoreType.DMA,
            pltpu.SemaphoreType.DMA,
            pltpu.SemaphoreType.DMA,
            pltpu.SemaphoreType.DMA,
            pltpu.SemaphoreType.DMA,
        ],
    )
    def agg_kernel(h_hbm, src_hbm, dst_hbm, out_hbm, acc, srcv, dstv, rows0,
                   rows1, rows2, isem, sem0, sem1, sem2, ssem0, ssem1, ssem2):
        c = lax.axis_index("c")
        s = lax.axis_index("s")
        w = c * _NSUB + s
        zeros = jnp.zeros((16,), jnp.float32)

        icp0 = pltpu.async_copy(src_hbm.at[w], srcv, isem)
        icp1 = pltpu.async_copy(dst_hbm.at[w], dstv, isem)

        @pl.when(c == 0)
        def _():
            pltpu.sync_copy(h_hbm.at[pl.ds(s * _RPT, _RPT)],
                            acc.at[pl.ds(s * _RPT, _RPT)])

            @pl.when(s == _NSUB - 1)
            def _():
                pltpu.sync_copy(h_hbm.at[pl.ds(_NSUB * _RPT,
                                               _N - _NSUB * _RPT)],
                                acc.at[pl.ds(_NSUB * _RPT,
                                             _N - _NSUB * _RPT)])

        @pl.when(c == 1)
        def _():
            @pl.loop(0, _ZROWS)
            def _(r):
                @pl.loop(0, _D, step=16)
                def _(j):
                    rows0[r, pl.ds(j, 16)] = zeros

            zsrc = rows0.at[pl.ds(0, _ZROWS)]

            @pl.loop(0, _RPT // _ZROWS)
            def _(j):
                pltpu.sync_copy(zsrc,
                                acc.at[pl.ds(s * _RPT + j * _ZROWS, _ZROWS)])

            @pl.when(s == _NSUB - 1)
            def _():
                pltpu.sync_copy(rows0.at[pl.ds(0, _N - _NSUB * _RPT)],
                                acc.at[pl.ds(_NSUB * _RPT,
                                             _N - _NSUB * _RPT)])

        bufs = (rows0, rows1, rows2)
        gsems = (sem0, sem1, sem2)

        def _gather(k, b):
            return pltpu.make_async_copy(
                h_hbm.at[srcv.at[0, pl.ds(k * _CHUNK, _CHUNK)]], bufs[b],
                gsems[b])

        def _scat(k, b, sem):
            return pltpu.async_copy(bufs[b], acc.at[dstv.at[k]], sem,
                                    add=True)

        def _scat_wait(k, b, sem):
            pltpu.make_async_copy(bufs[b], acc.at[dstv.at[k]], sem).wait()

        icp0.wait()
        icp1.wait()
        _gather(0, 0).start()
        plsc.subcore_barrier()

        _gather(1, 1).start()
        _gather(0, 0).wait()
        _scat(0, 0, ssem0)
        _gather(2, 2).start()
        _gather(1, 1).wait()
        _scat(1, 1, ssem1)
        _scat_wait(0, 0, ssem0)
        _gather(3, 0).start()
        _gather(2, 2).wait()
        _scat(2, 2, ssem2)

        @pl.loop(0, (_CPT - 5) // 3)
        def _(i):
            k = 3 * i + 3
            _scat_wait(k - 2, 1, ssem1)
            _gather(k + 1, 1).start()
            _gather(k, 0).wait()
            _scat(k, 0, ssem0)
            _scat_wait(k - 1, 2, ssem2)
            _gather(k + 2, 2).start()
            _gather(k + 1, 1).wait()
            _scat(k + 1, 1, ssem1)
            _scat_wait(k, 0, ssem0)
            _gather(k + 3, 0).start()
            _gather(k + 2, 2).wait()
            _scat(k + 2, 2, ssem2)

        _scat_wait(_CPT - 4, 1, ssem1)
        _gather(_CPT - 1, 1).start()
        _gather(_CPT - 2, 0).wait()
        _scat(_CPT - 2, 0, ssem0)
        _scat_wait(_CPT - 3, 2, ssem2)
        _gather(_CPT - 1, 1).wait()
        _scat(_CPT - 1, 1, ssem1)
        _scat_wait(_CPT - 2, 0, ssem0)
        _scat_wait(_CPT - 1, 1, ssem1)

        plsc.subcore_barrier()

        pltpu.sync_copy(acc.at[pl.ds(s * _RPT, _RPT)],
                        out_hbm.at[pl.ds(c * _N + s * _RPT, _RPT)])

        @pl.when(s == _NSUB - 1)
        def _():
            pltpu.sync_copy(acc.at[pl.ds(_NSUB * _RPT, _N - _NSUB * _RPT)],
                            out_hbm.at[pl.ds(c * _N + _NSUB * _RPT,
                                             _N - _NSUB * _RPT)])

    return agg_kernel(h, src3d, dst3d)


_BLK = 2000
_NB = _N // _BLK


def _tc_layer_body(a0_ref, a1_ref, wa_ref, ba_ref, g_ref,
                   be_ref, wb_ref, bb_ref, out_ref):
    h = a0_ref[...] + a1_ref[...]
    t = lax.dot_general(h, wa_ref[...], (((1,), (0,)), ((), ())),
                        precision=lax.Precision.DEFAULT,
                        preferred_element_type=jnp.float32)
    t = t + ba_ref[...]
    t = t * (g_ref[...] * (1.0 / (1.0 + _BN_EPS) ** 0.5)) + be_ref[...]
    t = jnp.maximum(t, 0.0)
    o = lax.dot_general(t, wb_ref[...], (((1,), (0,)), ((), ())),
                        precision=lax.Precision.DEFAULT,
                        preferred_element_type=jnp.float32)
    out_ref[...] = jnp.maximum(o + bb_ref[...], 0.0)


def _tc_layer(agg, wa, ba, g, be, wb, bb):
    return pl.pallas_call(
        _tc_layer_body,
        grid=(_NB,),
        in_specs=[
            pl.BlockSpec((_BLK, _D), lambda i: (i, 0)),
            pl.BlockSpec((_BLK, _D), lambda i: (i + _NB, 0)),
            pl.BlockSpec((_D, _D), lambda i: (0, 0)),
            pl.BlockSpec((1, _D), lambda i: (0, 0)),
            pl.BlockSpec((1, _D), lambda i: (0, 0)),
            pl.BlockSpec((1, _D), lambda i: (0, 0)),
            pl.BlockSpec((_D, _D), lambda i: (0, 0)),
            pl.BlockSpec((1, _D), lambda i: (0, 0)),
        ],
        out_specs=pl.BlockSpec((_BLK, _D), lambda i: (i, 0)),
        out_shape=jax.ShapeDtypeStruct((_N, _D), jnp.float32),
    )(agg, agg, wa, ba.reshape(1, _D), g.reshape(1, _D),
      be.reshape(1, _D), wb, bb.reshape(1, _D))


def _pool_contrib(h_blk, seg):
    onehot = (seg[:, None] == lax.broadcasted_iota(jnp.int32, (1, _NG), 1)
              ).astype(jnp.float32)
    return lax.dot_general(onehot, h_blk, (((0,), (0,)), ((), ())),
                           precision=lax.Precision.DEFAULT,
                           preferred_element_type=jnp.float32)


def _tc_pool_body(h_ref, b_ref, pool_ref):
    contrib = _pool_contrib(h_ref[...], b_ref[0, 0, :])

    @pl.when(pl.program_id(0) == 0)
    def _():
        pool_ref[...] = contrib

    @pl.when(pl.program_id(0) > 0)
    def _():
        pool_ref[...] += contrib


def _tc_pool(h, batch3d):
    return pl.pallas_call(
        _tc_pool_body,
        grid=(_NB,),
        in_specs=[
            pl.BlockSpec((_BLK, _D), lambda i: (i, 0)),
            pl.BlockSpec((1, 1, _BLK), lambda i: (i, 0, 0)),
        ],
        out_specs=pl.BlockSpec((_NG, _D), lambda i: (0, 0)),
        out_shape=jax.ShapeDtypeStruct((_NG, _D), jnp.float32),
    )(h, batch3d)


def _pool3_head_body(h_ref, b_ref, p1_ref, p2_ref, w1_ref, b1_ref, w2_ref,
                     b2_ref, out_ref, p3_ref):
    contrib = _pool_contrib(h_ref[...], b_ref[0, 0, :])

    @pl.when(pl.program_id(0) == 0)
    def _():
        p3_ref[...] = contrib

    @pl.when(pl.program_id(0) > 0)
    def _():
        p3_ref[...] += contrib

    @pl.when(pl.program_id(0) == _NB - 1)
    def _():
        acc = lax.dot_general(p1_ref[...], w1_ref[0:_D, :],
                              (((1,), (0,)), ((), ())),
                              precision=lax.Precision.DEFAULT,
                              preferred_element_type=jnp.float32)
        acc += lax.dot_general(p2_ref[...], w1_ref[_D:2 * _D, :],
                               (((1,), (0,)), ((), ())),
                               precision=lax.Precision.DEFAULT,
                               preferred_element_type=jnp.float32)
        acc += lax.dot_general(p3_ref[...], w1_ref[2 * _D:3 * _D, :],
                               (((1,), (0,)), ((), ())),
                               precision=lax.Precision.DEFAULT,
                               preferred_element_type=jnp.float32)
        hh = jnp.maximum(acc + b1_ref[...], 0.0)
        z = lax.dot_general(hh, w2_ref[...], (((1,), (0,)), ((), ())),
                            precision=lax.Precision.DEFAULT,
                            preferred_element_type=jnp.float32)
        z = z + b2_ref[...]
        m = jnp.max(z, axis=1, keepdims=True)
        lse = jnp.log(jnp.sum(jnp.exp(z - m), axis=1, keepdims=True)) + m
        out_ref[...] = z - lse


def _pool3_head(h3, batch3d, p1, p2, w1, b1, w2, b2):
    return pl.pallas_call(
        _pool3_head_body,
        grid=(_NB,),
        in_specs=[
            pl.BlockSpec((_BLK, _D), lambda i: (i, 0)),
            pl.BlockSpec((1, 1, _BLK), lambda i: (i, 0, 0)),
            pl.BlockSpec((_NG, _D), lambda i: (0, 0)),
            pl.BlockSpec((_NG, _D), lambda i: (0, 0)),
            pl.BlockSpec((3 * _D, 3 * _D), lambda i: (0, 0)),
            pl.BlockSpec((1, 3 * _D), lambda i: (0, 0)),
            pl.BlockSpec((3 * _D, _NC), lambda i: (0, 0)),
            pl.BlockSpec((1, _NC), lambda i: (0, 0)),
        ],
        out_specs=pl.BlockSpec((_NG, _NC), lambda i: (0, 0)),
        out_shape=jax.ShapeDtypeStruct((_NG, _NC), jnp.float32),
        scratch_shapes=[pltpu.VMEM((_NG, _D), jnp.float32)],
    )(h3, batch3d, p1, p2, w1, b1.reshape(1, 3 * _D), w2,
      b2.reshape(1, _NC))


def kernel(x, edge_index, batch, W1a, b1a, g1, be1, W1b, b1b, W2a, b2a, g2,
           be2, W2b, b2b, W3a, b3a, g3, be3, W3b, b3b, lin1_W, lin1_b,
           lin2_W, lin2_b):
    src2d = edge_index[0].reshape(_NW, 1, _CPT * _CHUNK)
    dst2d = edge_index[1].reshape(_NW, _CPT, _CHUNK)
    batch3d = batch.reshape(_NB, 1, _BLK)

    a1 = _sc_agg(x, src2d, dst2d)
    h1 = _tc_layer(a1, W1a, b1a, g1, be1, W1b, b1b)
    a2 = _sc_agg(h1, src2d, dst2d)
    p1 = _tc_pool(h1, batch3d)
    h2 = _tc_layer(a2, W2a, b2a, g2, be2, W2b, b2b)
    a3 = _sc_agg(h2, src2d, dst2d)
    p2 = _tc_pool(h2, batch3d)
    h3 = _tc_layer(a3, W3a, b3a, g3, be3, W3b, b3b)
    return _pool3_head(h3, batch3d, p1, p2, lin1_W, lin1_b, lin2_W, lin2_b)


# speedup vs baseline: 1.0169x; 1.0169x over previous
import functools

import jax
import jax.numpy as jnp
from jax import lax
from jax.experimental import pallas as pl
from jax.experimental.pallas import tpu as pltpu
from jax.experimental.pallas import tpu_sc as plsc

_N = 10000
_E = 320000
_D = 128
_NG = 128
_NC = 10
_BN_EPS = 1e-5

_NCORE = 2
_NSUB = 16
_NW = _NCORE * _NSUB
_CHUNK = 80
_CPT = _E // _CHUNK // _NW
_RPT = 624
_ZROWS = 48


def _sc_agg(h, src3d, dst3d):
    mesh = plsc.VectorSubcoreMesh(core_axis_name="c", subcore_axis_name="s")

    @functools.partial(
        pl.kernel,
        out_type=jax.ShapeDtypeStruct((2 * _N, _D), jnp.float32),
        mesh=mesh,
        compiler_params=pltpu.CompilerParams(use_tc_tiling_on_sc=False),
        scratch_types=[
            pltpu.VMEM_SHARED((_N, _D), jnp.float32),
            pltpu.VMEM((1, _CPT * _CHUNK), jnp.int32),
            pltpu.VMEM((_CPT, _CHUNK), jnp.int32),
            pltpu.VMEM((_CHUNK, _D), jnp.float32),
            pltpu.VMEM((_CHUNK, _D), jnp.float32),
            pltpu.VMEM((_CHUNK, _D), jnp.float32),
            pltpu.SemaphoreType.DMA,
            pltpu.SemaphoreType.DMA,
            pltpu.SemaphoreType.DMA,
            pltpu.SemaphoreType.DMA,
        ],
    )
    def agg_kernel(h_hbm, src_hbm, dst_hbm, out_hbm, acc, srcv, dstv, rows0,
                   rows1, rows2, isem, sem0, sem1, sem2):
        c = lax.axis_index("c")
        s = lax.axis_index("s")
        w = c * _NSUB + s
        zeros = jnp.zeros((16,), jnp.float32)

        icp0 = pltpu.async_copy(src_hbm.at[w], srcv, isem)
        icp1 = pltpu.async_copy(dst_hbm.at[w], dstv, isem)

        @pl.loop(0, _ZROWS)
        def _(r):
            @pl.loop(0, _D, step=16)
            def _(j):
                rows0[r, pl.ds(j, 16)] = zeros

        zsrc = rows0.at[pl.ds(0, _ZROWS)]

        @pl.loop(0, _RPT // _ZROWS)
        def _(j):
            pltpu.sync_copy(zsrc, acc.at[pl.ds(s * _RPT + j * _ZROWS, _ZROWS)])

        @pl.when(s == _NSUB - 1)
        def _():
            pltpu.sync_copy(rows0.at[pl.ds(0, _N - _NSUB * _RPT)],
                            acc.at[pl.ds(_NSUB * _RPT, _N - _NSUB * _RPT)])

        def _gather(k, buf, sem):
            return pltpu.make_async_copy(
                h_hbm.at[srcv.at[0, pl.ds(k * _CHUNK, _CHUNK)]], buf, sem)

        icp0.wait()
        icp1.wait()
        _gather(0, rows0, sem0).start()
        _gather(1, rows1, sem1).start()
        plsc.subcore_barrier()

        @pl.loop(0, _CPT // 3)
        def _(i):
            k = 3 * i
            _gather(k, rows0, sem0).wait()
            _gather(k + 2, rows2, sem2).start()
            pltpu.sync_copy(rows0, acc.at[dstv.at[k]], add=True)
            _gather(k + 1, rows1, sem1).wait()
            _gather(k + 3, rows0, sem0).start()
            pltpu.sync_copy(rows1, acc.at[dstv.at[k + 1]], add=True)
            _gather(k + 2, rows2, sem2).wait()
            _gather(k + 4, rows1, sem1).start()
            pltpu.sync_copy(rows2, acc.at[dstv.at[k + 2]], add=True)

        _gather(_CPT - 2, rows0, sem0).wait()
        pltpu.sync_copy(rows0, acc.at[dstv.at[_CPT - 2]], add=True)
        _gather(_CPT - 1, rows1, sem1).wait()
        pltpu.sync_copy(rows1, acc.at[dstv.at[_CPT - 1]], add=True)

        plsc.subcore_barrier()

        pltpu.sync_copy(acc.at[pl.ds(s * _RPT, _RPT)],
                        out_hbm.at[pl.ds(c * _N + s * _RPT, _RPT)])

        @pl.when(s == _NSUB - 1)
        def _():
            pltpu.sync_copy(acc.at[pl.ds(_NSUB * _RPT, _N - _NSUB * _RPT)],
                            out_hbm.at[pl.ds(c * _N + _NSUB * _RPT,
                                             _N - _NSUB * _RPT)])

    return agg_kernel(h, src3d, dst3d)


_BLK = 2000
_NB = _N // _BLK


def _tc_layer_body(hp_ref, a0_ref, a1_ref, wa_ref, ba_ref, g_ref,
                   be_ref, wb_ref, bb_ref, out_ref):
    h = hp_ref[...] + a0_ref[...] + a1_ref[...]
    t = lax.dot_general(h, wa_ref[...], (((1,), (0,)), ((), ())),
                        precision=lax.Precision.DEFAULT,
                        preferred_element_type=jnp.float32)
    t = t + ba_ref[...]
    t = t * (g_ref[...] * (1.0 / (1.0 + _BN_EPS) ** 0.5)) + be_ref[...]
    t = jnp.maximum(t, 0.0)
    o = lax.dot_general(t, wb_ref[...], (((1,), (0,)), ((), ())),
                        precision=lax.Precision.DEFAULT,
                        preferred_element_type=jnp.float32)
    out_ref[...] = jnp.maximum(o + bb_ref[...], 0.0)


def _tc_layer(hp, agg, wa, ba, g, be, wb, bb):
    return pl.pallas_call(
        _tc_layer_body,
        grid=(_NB,),
        in_specs=[
            pl.BlockSpec((_BLK, _D), lambda i: (i, 0)),
            pl.BlockSpec((_BLK, _D), lambda i: (i, 0)),
            pl.BlockSpec((_BLK, _D), lambda i: (i + _NB, 0)),
            pl.BlockSpec((_D, _D), lambda i: (0, 0)),
            pl.BlockSpec((1, _D), lambda i: (0, 0)),
            pl.BlockSpec((1, _D), lambda i: (0, 0)),
            pl.BlockSpec((1, _D), lambda i: (0, 0)),
            pl.BlockSpec((_D, _D), lambda i: (0, 0)),
            pl.BlockSpec((1, _D), lambda i: (0, 0)),
        ],
        out_specs=pl.BlockSpec((_BLK, _D), lambda i: (i, 0)),
        out_shape=jax.ShapeDtypeStruct((_N, _D), jnp.float32),
    )(hp, agg, agg, wa, ba.reshape(1, _D), g.reshape(1, _D),
      be.reshape(1, _D), wb, bb.reshape(1, _D))


def _pool_contrib(h_blk, seg):
    onehot = (seg[:, None] == lax.broadcasted_iota(jnp.int32, (1, _NG), 1)
              ).astype(jnp.float32)
    return lax.dot_general(onehot, h_blk, (((0,), (0,)), ((), ())),
                           precision=lax.Precision.DEFAULT,
                           preferred_element_type=jnp.float32)


def _tc_pool_body(h_ref, b_ref, pool_ref):
    contrib = _pool_contrib(h_ref[...], b_ref[0, 0, :])

    @pl.when(pl.program_id(0) == 0)
    def _():
        pool_ref[...] = contrib

    @pl.when(pl.program_id(0) > 0)
    def _():
        pool_ref[...] += contrib


def _tc_pool(h, batch3d):
    return pl.pallas_call(
        _tc_pool_body,
        grid=(_NB,),
        in_specs=[
            pl.BlockSpec((_BLK, _D), lambda i: (i, 0)),
            pl.BlockSpec((1, 1, _BLK), lambda i: (i, 0, 0)),
        ],
        out_specs=pl.BlockSpec((_NG, _D), lambda i: (0, 0)),
        out_shape=jax.ShapeDtypeStruct((_NG, _D), jnp.float32),
    )(h, batch3d)


def _pool3_head_body(h_ref, b_ref, p1_ref, p2_ref, w1_ref, b1_ref, w2_ref,
                     b2_ref, out_ref, p3_ref):
    contrib = _pool_contrib(h_ref[...], b_ref[0, 0, :])

    @pl.when(pl.program_id(0) == 0)
    def _():
        p3_ref[...] = contrib

    @pl.when(pl.program_id(0) > 0)
    def _():
        p3_ref[...] += contrib

    @pl.when(pl.program_id(0) == _NB - 1)
    def _():
        acc = lax.dot_general(p1_ref[...], w1_ref[0:_D, :],
                              (((1,), (0,)), ((), ())),
                              precision=lax.Precision.DEFAULT,
                              preferred_element_type=jnp.float32)
        acc += lax.dot_general(p2_ref[...], w1_ref[_D:2 * _D, :],
                               (((1,), (0,)), ((), ())),
                               precision=lax.Precision.DEFAULT,
                               preferred_element_type=jnp.float32)
        acc += lax.dot_general(p3_ref[...], w1_ref[2 * _D:3 * _D, :],
                               (((1,), (0,)), ((), ())),
                               precision=lax.Precision.DEFAULT,
                               preferred_element_type=jnp.float32)
        hh = jnp.maximum(acc + b1_ref[...], 0.0)
        z = lax.dot_general(hh, w2_ref[...], (((1,), (0,)), ((), ())),
                            precision=lax.Precision.DEFAULT,
                            preferred_element_type=jnp.float32)
        z = z + b2_ref[...]
        m = jnp.max(z, axis=1, keepdims=True)
        lse = jnp.log(jnp.sum(jnp.exp(z - m), axis=1, keepdims=True)) + m
        out_ref[...] = z - lse


def _pool3_head(h3, batch3d, p1, p2, w1, b1, w2, b2):
    return pl.pallas_call(
        _pool3_head_body,
        grid=(_NB,),
        in_specs=[
            pl.BlockSpec((_BLK, _D), lambda i: (i, 0)),
            pl.BlockSpec((1, 1, _BLK), lambda i: (i, 0, 0)),
            pl.BlockSpec((_NG, _D), lambda i: (0, 0)),
            pl.BlockSpec((_NG, _D), lambda i: (0, 0)),
            pl.BlockSpec((3 * _D, 3 * _D), lambda i: (0, 0)),
            pl.BlockSpec((1, 3 * _D), lambda i: (0, 0)),
            pl.BlockSpec((3 * _D, _NC), lambda i: (0, 0)),
            pl.BlockSpec((1, _NC), lambda i: (0, 0)),
        ],
        out_specs=pl.BlockSpec((_NG, _NC), lambda i: (0, 0)),
        out_shape=jax.ShapeDtypeStruct((_NG, _NC), jnp.float32),
        scratch_shapes=[pltpu.VMEM((_NG, _D), jnp.float32)],
    )(h3, batch3d, p1, p2, w1, b1.reshape(1, 3 * _D), w2,
      b2.reshape(1, _NC))


def kernel(x, edge_index, batch, W1a, b1a, g1, be1, W1b, b1b, W2a, b2a, g2,
           be2, W2b, b2b, W3a, b3a, g3, be3, W3b, b3b, lin1_W, lin1_b,
           lin2_W, lin2_b):
    src2d = edge_index[0].reshape(_NW, 1, _CPT * _CHUNK)
    dst2d = edge_index[1].reshape(_NW, _CPT, _CHUNK)
    batch3d = batch.reshape(_NB, 1, _BLK)

    a1 = _sc_agg(x, src2d, dst2d)
    h1 = _tc_layer(x, a1, W1a, b1a, g1, be1, W1b, b1b)
    a2 = _sc_agg(h1, src2d, dst2d)
    p1 = _tc_pool(h1, batch3d)
    h2 = _tc_layer(h1, a2, W2a, b2a, g2, be2, W2b, b2b)
    a3 = _sc_agg(h2, src2d, dst2d)
    p2 = _tc_pool(h2, batch3d)
    h3 = _tc_layer(h2, a3, W3a, b3a, g3, be3, W3b, b3b)
    return _pool3_head(h3, batch3d, p1, p2, lin1_W, lin1_b, lin2_W, lin2_b)


# speedup vs baseline: 1.0365x; 1.0193x over previous
import functools

import jax
import jax.numpy as jnp
from jax import lax
from jax.experimental import pallas as pl
from jax.experimental.pallas import tpu as pltpu
from jax.experimental.pallas import tpu_sc as plsc

_N = 10000
_E = 320000
_D = 128
_NG = 128
_NC = 10
_BN_EPS = 1e-5

_NCORE = 2
_NSUB = 16
_NW = _NCORE * _NSUB
_CHUNK = 80
_CPT = _E // _CHUNK // _NW
_RPT = 624
_ZROWS = 48


def _sc_agg(h, src3d, dst3d):
    mesh = plsc.VectorSubcoreMesh(core_axis_name="c", subcore_axis_name="s")

    @functools.partial(
        pl.kernel,
        out_type=jax.ShapeDtypeStruct((2 * _N, _D), jnp.float32),
        mesh=mesh,
        compiler_params=pltpu.CompilerParams(use_tc_tiling_on_sc=False),
        scratch_types=[
            pltpu.VMEM_SHARED((_N, _D), jnp.float32),
            pltpu.VMEM((1, _CPT * _CHUNK), jnp.int32),
            pltpu.VMEM((_CPT, _CHUNK), jnp.int32),
            pltpu.VMEM((_CHUNK, _D), jnp.float32),
            pltpu.VMEM((_CHUNK, _D), jnp.float32),
            pltpu.VMEM((_CHUNK, _D), jnp.float32),
            pltpu.SemaphoreType.DMA,
            pltpu.SemaphoreType.DMA,
            pltpu.SemaphoreType.DMA,
            pltpu.SemaphoreType.DMA,
        ],
    )
    def agg_kernel(h_hbm, src_hbm, dst_hbm, out_hbm, acc, srcv, dstv, rows0,
                   rows1, rows2, isem, sem0, sem1, sem2):
        c = lax.axis_index("c")
        s = lax.axis_index("s")
        w = c * _NSUB + s
        zeros = jnp.zeros((16,), jnp.float32)

        icp0 = pltpu.async_copy(src_hbm.at[w], srcv, isem)
        icp1 = pltpu.async_copy(dst_hbm.at[w], dstv, isem)

        @pl.loop(0, _ZROWS)
        def _(r):
            @pl.loop(0, _D, step=16)
            def _(j):
                rows0[r, pl.ds(j, 16)] = zeros

        zsrc = rows0.at[pl.ds(0, _ZROWS)]

        @pl.loop(0, _RPT // _ZROWS)
        def _(j):
            pltpu.sync_copy(zsrc, acc.at[pl.ds(s * _RPT + j * _ZROWS, _ZROWS)])

        @pl.when(s == _NSUB - 1)
        def _():
            pltpu.sync_copy(rows0.at[pl.ds(0, _N - _NSUB * _RPT)],
                            acc.at[pl.ds(_NSUB * _RPT, _N - _NSUB * _RPT)])

        def _gather(k, buf, sem):
            return pltpu.make_async_copy(
                h_hbm.at[srcv.at[0, pl.ds(k * _CHUNK, _CHUNK)]], buf, sem)

        icp0.wait()
        icp1.wait()
        _gather(0, rows0, sem0).start()
        _gather(1, rows1, sem1).start()
        plsc.subcore_barrier()

        @pl.loop(0, _CPT // 3)
        def _(i):
            k = 3 * i
            _gather(k, rows0, sem0).wait()
            _gather(k + 2, rows2, sem2).start()
            pltpu.sync_copy(rows0, acc.at[dstv.at[k]], add=True)
            _gather(k + 1, rows1, sem1).wait()
            _gather(k + 3, rows0, sem0).start()
            pltpu.sync_copy(rows1, acc.at[dstv.at[k + 1]], add=True)
            _gather(k + 2, rows2, sem2).wait()
            _gather(k + 4, rows1, sem1).start()
            pltpu.sync_copy(rows2, acc.at[dstv.at[k + 2]], add=True)

        _gather(_CPT - 2, rows0, sem0).wait()
        pltpu.sync_copy(rows0, acc.at[dstv.at[_CPT - 2]], add=True)
        _gather(_CPT - 1, rows1, sem1).wait()
        pltpu.sync_copy(rows1, acc.at[dstv.at[_CPT - 1]], add=True)

        plsc.subcore_barrier()

        pltpu.sync_copy(acc.at[pl.ds(s * _RPT, _RPT)],
                        out_hbm.at[pl.ds(c * _N + s * _RPT, _RPT)])

        @pl.when(s == _NSUB - 1)
        def _():
            pltpu.sync_copy(acc.at[pl.ds(_NSUB * _RPT, _N - _NSUB * _RPT)],
                            out_hbm.at[pl.ds(c * _N + _NSUB * _RPT,
                                             _N - _NSUB * _RPT)])

    return agg_kernel(h, src3d, dst3d)


_BLK = 2000
_NB = _N // _BLK


def _tc_layer_body(hp_ref, a0_ref, a1_ref, wa_ref, ba_ref, g_ref,
                   be_ref, wb_ref, bb_ref, out_ref):
    h = hp_ref[...] + a0_ref[...] + a1_ref[...]
    t = lax.dot_general(h, wa_ref[...], (((1,), (0,)), ((), ())),
                        precision=lax.Precision.DEFAULT,
                        preferred_element_type=jnp.float32)
    t = t + ba_ref[...]
    t = t * (g_ref[...] * (1.0 / (1.0 + _BN_EPS) ** 0.5)) + be_ref[...]
    t = jnp.maximum(t, 0.0)
    o = lax.dot_general(t, wb_ref[...], (((1,), (0,)), ((), ())),
                        precision=lax.Precision.DEFAULT,
                        preferred_element_type=jnp.float32)
    out_ref[...] = jnp.maximum(o + bb_ref[...], 0.0)


def _tc_layer(hp, agg, wa, ba, g, be, wb, bb):
    return pl.pallas_call(
        _tc_layer_body,
        grid=(_NB,),
        in_specs=[
            pl.BlockSpec((_BLK, _D), lambda i: (i, 0)),
            pl.BlockSpec((_BLK, _D), lambda i: (i, 0)),
            pl.BlockSpec((_BLK, _D), lambda i: (i + _NB, 0)),
            pl.BlockSpec((_D, _D), lambda i: (0, 0)),
            pl.BlockSpec((1, _D), lambda i: (0, 0)),
            pl.BlockSpec((1, _D), lambda i: (0, 0)),
            pl.BlockSpec((1, _D), lambda i: (0, 0)),
            pl.BlockSpec((_D, _D), lambda i: (0, 0)),
            pl.BlockSpec((1, _D), lambda i: (0, 0)),
        ],
        out_specs=pl.BlockSpec((_BLK, _D), lambda i: (i, 0)),
        out_shape=jax.ShapeDtypeStruct((_N, _D), jnp.float32),
    )(hp, agg, agg, wa, ba.reshape(1, _D), g.reshape(1, _D),
      be.reshape(1, _D), wb, bb.reshape(1, _D))


def _pool_contrib(h_blk, seg):
    onehot = (seg[:, None] == lax.broadcasted_iota(jnp.int32, (1, _NG), 1)
              ).astype(jnp.float32)
    return lax.dot_general(onehot, h_blk, (((0,), (0,)), ((), ())),
                           precision=lax.Precision.DEFAULT,
                           preferred_element_type=jnp.float32)


def _tc_pool_body(h_ref, b_ref, pool_ref):
    contrib = _pool_contrib(h_ref[...], b_ref[0, 0, :])

    @pl.when(pl.program_id(0) == 0)
    def _():
        pool_ref[...] = contrib

    @pl.when(pl.program_id(0) > 0)
    def _():
        pool_ref[...] += contrib


def _tc_pool(h, batch3d):
    return pl.pallas_call(
        _tc_pool_body,
        grid=(_NB,),
        in_specs=[
            pl.BlockSpec((_BLK, _D), lambda i: (i, 0)),
            pl.BlockSpec((1, 1, _BLK), lambda i: (i, 0, 0)),
        ],
        out_specs=pl.BlockSpec((_NG, _D), lambda i: (0, 0)),
        out_shape=jax.ShapeDtypeStruct((_NG, _D), jnp.float32),
    )(h, batch3d)


def _pool3_head_body(hp_ref, a0_ref, a1_ref, wa_ref, ba_ref, g_ref, be_ref,
                     wb_ref, bb_ref, b_ref, p1_ref, p2_ref, w1_ref, b1_ref,
                     w2_ref, b2_ref, out_ref, p3_ref):
    h = hp_ref[...] + a0_ref[...] + a1_ref[...]
    t = lax.dot_general(h, wa_ref[...], (((1,), (0,)), ((), ())),
                        precision=lax.Precision.DEFAULT,
                        preferred_element_type=jnp.float32)
    t = t + ba_ref[...]
    t = t * (g_ref[...] * (1.0 / (1.0 + _BN_EPS) ** 0.5)) + be_ref[...]
    t = jnp.maximum(t, 0.0)
    o = lax.dot_general(t, wb_ref[...], (((1,), (0,)), ((), ())),
                        precision=lax.Precision.DEFAULT,
                        preferred_element_type=jnp.float32)
    o = jnp.maximum(o + bb_ref[...], 0.0)
    contrib = _pool_contrib(o, b_ref[0, 0, :])

    @pl.when(pl.program_id(0) == 0)
    def _():
        p3_ref[...] = contrib

    @pl.when(pl.program_id(0) > 0)
    def _():
        p3_ref[...] += contrib

    @pl.when(pl.program_id(0) == _NB - 1)
    def _():
        acc = lax.dot_general(p1_ref[...], w1_ref[0:_D, :],
                              (((1,), (0,)), ((), ())),
                              precision=lax.Precision.DEFAULT,
                              preferred_element_type=jnp.float32)
        acc += lax.dot_general(p2_ref[...], w1_ref[_D:2 * _D, :],
                               (((1,), (0,)), ((), ())),
                               precision=lax.Precision.DEFAULT,
                               preferred_element_type=jnp.float32)
        acc += lax.dot_general(p3_ref[...], w1_ref[2 * _D:3 * _D, :],
                               (((1,), (0,)), ((), ())),
                               precision=lax.Precision.DEFAULT,
                               preferred_element_type=jnp.float32)
        hh = jnp.maximum(acc + b1_ref[...], 0.0)
        z = lax.dot_general(hh, w2_ref[...], (((1,), (0,)), ((), ())),
                            precision=lax.Precision.DEFAULT,
                            preferred_element_type=jnp.float32)
        z = z + b2_ref[...]
        m = jnp.max(z, axis=1, keepdims=True)
        lse = jnp.log(jnp.sum(jnp.exp(z - m), axis=1, keepdims=True)) + m
        out_ref[...] = z - lse


def _pool3_head(hp, agg, wa, ba, g, be, wb, bb, batch3d, p1, p2, w1, b1,
                w2, b2):
    return pl.pallas_call(
        _pool3_head_body,
        grid=(_NB,),
        in_specs=[
            pl.BlockSpec((_BLK, _D), lambda i: (i, 0)),
            pl.BlockSpec((_BLK, _D), lambda i: (i, 0)),
            pl.BlockSpec((_BLK, _D), lambda i: (i + _NB, 0)),
            pl.BlockSpec((_D, _D), lambda i: (0, 0)),
            pl.BlockSpec((1, _D), lambda i: (0, 0)),
            pl.BlockSpec((1, _D), lambda i: (0, 0)),
            pl.BlockSpec((1, _D), lambda i: (0, 0)),
            pl.BlockSpec((_D, _D), lambda i: (0, 0)),
            pl.BlockSpec((1, _D), lambda i: (0, 0)),
            pl.BlockSpec((1, 1, _BLK), lambda i: (i, 0, 0)),
            pl.BlockSpec((_NG, _D), lambda i: (0, 0)),
            pl.BlockSpec((_NG, _D), lambda i: (0, 0)),
            pl.BlockSpec((3 * _D, 3 * _D), lambda i: (0, 0)),
            pl.BlockSpec((1, 3 * _D), lambda i: (0, 0)),
            pl.BlockSpec((3 * _D, _NC), lambda i: (0, 0)),
            pl.BlockSpec((1, _NC), lambda i: (0, 0)),
        ],
        out_specs=pl.BlockSpec((_NG, _NC), lambda i: (0, 0)),
        out_shape=jax.ShapeDtypeStruct((_NG, _NC), jnp.float32),
        scratch_shapes=[pltpu.VMEM((_NG, _D), jnp.float32)],
    )(hp, agg, agg, wa, ba.reshape(1, _D), g.reshape(1, _D),
      be.reshape(1, _D), wb, bb.reshape(1, _D), batch3d, p1, p2, w1,
      b1.reshape(1, 3 * _D), w2, b2.reshape(1, _NC))


def kernel(x, edge_index, batch, W1a, b1a, g1, be1, W1b, b1b, W2a, b2a, g2,
           be2, W2b, b2b, W3a, b3a, g3, be3, W3b, b3b, lin1_W, lin1_b,
           lin2_W, lin2_b):
    src2d = edge_index[0].reshape(_NW, 1, _CPT * _CHUNK)
    dst2d = edge_index[1].reshape(_NW, _CPT, _CHUNK)
    batch3d = batch.reshape(_NB, 1, _BLK)

    a1 = _sc_agg(x, src2d, dst2d)
    h1 = _tc_layer(x, a1, W1a, b1a, g1, be1, W1b, b1b)
    a2 = _sc_agg(h1, src2d, dst2d)
    p1 = _tc_pool(h1, batch3d)
    h2 = _tc_layer(h1, a2, W2a, b2a, g2, be2, W2b, b2b)
    a3 = _sc_agg(h2, src2d, dst2d)
    p2 = _tc_pool(h2, batch3d)
    return _pool3_head(h2, a3, W3a, b3a, g3, be3, W3b, b3b, batch3d, p1,
                       p2, lin1_W, lin1_b, lin2_W, lin2_b)
